# baseline (speedup 1.0000x reference)
import functools

import numpy as np
import jax
import jax.numpy as jnp
from jax import lax
from jax.experimental import pallas as pl
from jax.experimental.pallas import tpu as pltpu, tpu_sc as plsc

SHAPES = ((8, 32, 32), (4, 16, 16), (2, 8, 8))
DM = 256
NH = 8
NL = 3
NP = 4
DFF = 1024
HD = DM // NH
LQ = sum(d * h * w for d, h, w in SHAPES)
NCOL = NH * NL * NP
NCORN = 8
BQ = 128
GRID = LQ // BQ
NWORK = 32
QPW = LQ // NWORK

_lvl = np.array([(k % 12) // 4 for k in range(NCOL)])
_m = np.array([k // 12 for k in range(NCOL)])
_w_np = np.array([SHAPES[l][2] for l in _lvl], np.float32)[None]
_h_np = np.array([SHAPES[l][1] for l in _lvl], np.float32)[None]
_d_np = np.array([SHAPES[l][0] for l in _lvl], np.float32)[None]
_starts = np.cumsum([0] + [d * h * w for d, h, w in SHAPES])[:3]
_lvlstart_np = np.array([_starts[l] for l in _lvl], np.float32)[None]
_m_np = _m.astype(np.float32)[None]
_S_np = np.kron(np.eye(NH, dtype=np.float32), np.ones((NL * NP, NL * NP), np.float32))

_ref_list = []
for d_, h_, w_ in SHAPES:
    zz, yy, xx = np.meshgrid(np.arange(d_) + 0.5, np.arange(h_) + 0.5,
                             np.arange(w_) + 0.5, indexing="ij")
    _ref_list.append(np.stack([(zz / d_).ravel(), (yy / h_).ravel(),
                               (xx / w_).ravel()], -1))
_refq_np = np.concatenate(_ref_list, 0).astype(np.float32)

_perm = np.empty(3 * NCOL, np.int64)
for m in range(NH):
    for l in range(NL):
        for p in range(NP):
            for c in range(3):
                _perm[c * NCOL + m * 12 + l * 4 + p] = ((m * 3 + l) * 4 + p) * 3 + c


def _tc1_body(out_ref, posle_ref, refq_ref, Wv_ref, bv_ref, Woff_ref, boff_ref,
              Wa_ref, ba_ref, S_ref, cst_ref, value_ref, idx_ref, wgt_ref):
    x = out_ref[...]
    q = x + posle_ref[...]
    value_ref[...] = jnp.dot(x, Wv_ref[...], preferred_element_type=jnp.float32) + bv_ref[...]

    offr = jnp.dot(q, Woff_ref[...], preferred_element_type=jnp.float32) + boff_ref[...]
    a = jnp.dot(q, Wa_ref[...], preferred_element_type=jnp.float32) + ba_ref[...]
    a = a - jnp.max(a, axis=-1, keepdims=True)
    e = jnp.exp(a)
    ssum = jnp.dot(e, S_ref[...], preferred_element_type=jnp.float32)
    aw = e / ssum

    w96 = cst_ref[0:1, :]
    h96 = cst_ref[1:2, :]
    d96 = cst_ref[2:3, :]
    lvlstart = cst_ref[3:4, :]
    m96 = cst_ref[4:5, :]

    r0 = refq_ref[:, 0:1]
    r1 = refq_ref[:, 1:2]
    r2 = refq_ref[:, 2:3]
    ix = r0 * w96 + offr[:, 0:NCOL] - 0.5
    iy = r1 * h96 + offr[:, NCOL:2 * NCOL] - 0.5
    iz = r2 * d96 + offr[:, 2 * NCOL:3 * NCOL] - 0.5
    x0 = jnp.floor(ix)
    y0 = jnp.floor(iy)
    z0 = jnp.floor(iz)

    idx_parts = []
    wgt_parts = []
    for dz in (0, 1):
        for dy in (0, 1):
            for dx in (0, 1):
                xi = x0 + dx
                yi = y0 + dy
                zi = z0 + dz
                wx = 1.0 - jnp.abs(ix - xi)
                wy = 1.0 - jnp.abs(iy - yi)
                wz = 1.0 - jnp.abs(iz - zi)
                valid = ((xi >= 0) & (xi <= w96 - 1) & (yi >= 0) & (yi <= h96 - 1)
                         & (zi >= 0) & (zi <= d96 - 1))
                w = wx * wy * wz * aw * valid.astype(jnp.float32)
                vox = (jnp.clip(zi, 0, d96 - 1) * h96 + jnp.clip(yi, 0, h96 - 1)) * w96 \
                    + jnp.clip(xi, 0, w96 - 1)
                row = (lvlstart + vox) * float(NH) + m96
                idx_parts.append(row.astype(jnp.int32)[:, None, :])
                wgt_parts.append(w[:, None, :])
    idx_ref[...] = jnp.concatenate(idx_parts, axis=1)
    wgt_ref[...] = jnp.concatenate(wgt_parts, axis=1)


_cst_np = np.concatenate([_w_np, _h_np, _d_np, _lvlstart_np, _m_np], 0)


def _tc1_call(out, posle, refq, Wv, bv, Woff_r, boff_r, Wa, ba):
    full = lambda shape: pl.BlockSpec(shape, lambda i: (0,) * len(shape))
    return pl.pallas_call(
        _tc1_body,
        grid=(GRID,),
        in_specs=[
            pl.BlockSpec((BQ, DM), lambda i: (i, 0)),
            pl.BlockSpec((BQ, DM), lambda i: (i, 0)),
            pl.BlockSpec((BQ, 3), lambda i: (i, 0)),
            full((DM, DM)), full((DM,)),
            full((DM, 3 * NCOL)), full((3 * NCOL,)),
            full((DM, NCOL)), full((NCOL,)),
            full((NCOL, NCOL)), full((5, NCOL)),
        ],
        out_specs=[
            pl.BlockSpec((BQ, DM), lambda i: (i, 0)),
            pl.BlockSpec((BQ, NCORN, NCOL), lambda i: (i, 0, 0)),
            pl.BlockSpec((BQ, NCORN, NCOL), lambda i: (i, 0, 0)),
        ],
        out_shape=[
            jax.ShapeDtypeStruct((LQ, DM), jnp.float32),
            jax.ShapeDtypeStruct((LQ, NCORN, NCOL), jnp.int32),
            jax.ShapeDtypeStruct((LQ, NCORN, NCOL), jnp.float32),
        ],
    )(out, posle, refq, Wv, bv, Woff_r, boff_r, Wa, ba,
      jnp.asarray(_S_np), jnp.asarray(_cst_np))


NROW = NCORN * NCOL


def _sc_body(value_hbm, idx_hbm, wgt_hbm, out_hbm,
             idx0, idx1, wgt0, wgt1, rows0, rows1, outv0, outv1,
             gs0, gs1, is0, is1, ws0, ws1, os0, os1):
    wid = lax.axis_index("s") * 2 + lax.axis_index("c")
    base = wid * QPW

    def prefetch(idxv, wgtv, isem, wsem, qq):
        pltpu.async_copy(idx_hbm.at[qq], idxv, isem)
        pltpu.async_copy(wgt_hbm.at[qq], wgtv.at[pl.ds(0, NROW)], wsem)

    def start_gather(idxv, isem, rowsv, gsem, qq):
        pltpu.make_async_copy(idx_hbm.at[qq], idxv, isem).wait()
        pltpu.async_copy(value_hbm.at[idxv], rowsv, gsem)

    def wait_gather(idxv, rowsv, gsem):
        pltpu.make_async_copy(value_hbm.at[idxv], rowsv, gsem).wait()

    def compute(wgtv, wsem, rowsv, outv, osem, qq, first):
        pltpu.make_async_copy(wgt_hbm.at[qq], wgtv.at[pl.ds(0, NROW)], wsem).wait()

        @pl.when(jnp.logical_not(first))
        def _():
            pltpu.make_async_copy(outv, out_hbm.at[qq], osem).wait()

        def cbody(c, accs):
            cb = c * NCOL
            new = []
            for m in range(NH):
                jb = cb + m * 12
                wvec = wgtv[pl.ds(jb, 16)]
                a0 = accs[2 * m]
                a1 = accs[2 * m + 1]
                for t in range(12):
                    w = wvec[t]
                    a0 = a0 + w * rowsv[jb + t, pl.ds(0, 16)]
                    a1 = a1 + w * rowsv[jb + t, pl.ds(16, 16)]
                new.append(a0)
                new.append(a1)
            return tuple(new)

        z = jnp.zeros((16,), jnp.float32)
        accs = lax.fori_loop(0, NCORN, cbody, (z,) * (2 * NH))
        for m in range(NH):
            outv[pl.ds(m * 32, 16)] = accs[2 * m]
            outv[pl.ds(m * 32 + 16, 16)] = accs[2 * m + 1]
        pltpu.async_copy(outv, out_hbm.at[qq], osem)

    prefetch(idx0, wgt0, is0, ws0, base)
    start_gather(idx0, is0, rows0, gs0, base)
    prefetch(idx1, wgt1, is1, ws1, base + 1)

    def ibody(i, carry):
        q0 = base + 2 * i
        first = i == 0
        not_last = i < QPW // 2 - 1
        wait_gather(idx0, rows0, gs0)
        start_gather(idx1, is1, rows1, gs1, q0 + 1)
        compute(wgt0, ws0, rows0, outv0, os0, q0, first)

        @pl.when(not_last)
        def _():
            prefetch(idx0, wgt0, is0, ws0, q0 + 2)

        wait_gather(idx1, rows1, gs1)
        compute(wgt1, ws1, rows1, outv1, os1, q0 + 1, first)

        @pl.when(not_last)
        def _():
            start_gather(idx0, is0, rows0, gs0, q0 + 2)
            prefetch(idx1, wgt1, is1, ws1, q0 + 3)

        return carry

    lax.fori_loop(0, QPW // 2, ibody, 0)
    pltpu.make_async_copy(outv0, out_hbm.at[base], os0).wait()
    pltpu.make_async_copy(outv1, out_hbm.at[base + 1], os1).wait()


@functools.partial(
    pl.kernel,
    out_type=jax.ShapeDtypeStruct((LQ, DM), jnp.float32),
    mesh=plsc.VectorSubcoreMesh(core_axis_name="c", subcore_axis_name="s",
                                num_cores=2, num_subcores=16),
    compiler_params=pltpu.CompilerParams(needs_layout_passes=False,
                                         use_tc_tiling_on_sc=False),
    scratch_types=[
        pltpu.VMEM((NROW,), jnp.int32),
        pltpu.VMEM((NROW,), jnp.int32),
        pltpu.VMEM((NROW + 16,), jnp.float32),
        pltpu.VMEM((NROW + 16,), jnp.float32),
        pltpu.VMEM((NROW, HD), jnp.float32),
        pltpu.VMEM((NROW, HD), jnp.float32),
        pltpu.VMEM((DM,), jnp.float32),
        pltpu.VMEM((DM,), jnp.float32),
        pltpu.SemaphoreType.DMA,
        pltpu.SemaphoreType.DMA,
        pltpu.SemaphoreType.DMA,
        pltpu.SemaphoreType.DMA,
        pltpu.SemaphoreType.DMA,
        pltpu.SemaphoreType.DMA,
        pltpu.SemaphoreType.DMA,
        pltpu.SemaphoreType.DMA,
    ],
)
def _sc_call(value_hbm, idx_hbm, wgt_hbm, out_hbm,
             idx0, idx1, wgt0, wgt1, rows0, rows1, outv0, outv1,
             gs0, gs1, is0, is1, ws0, ws1, os0, os1):
    _sc_body(value_hbm, idx_hbm, wgt_hbm, out_hbm,
             idx0, idx1, wgt0, wgt1, rows0, rows1, outv0, outv1,
             gs0, gs1, is0, is1, ws0, ws1, os0, os1)


def _ln(x, g, b):
    m = jnp.mean(x, axis=-1, keepdims=True)
    v = jnp.mean((x - m) ** 2, axis=-1, keepdims=True)
    return (x - m) * lax.rsqrt(v + 1e-5) * g + b


def _tc2_body(out_ref, attn_ref, Wo_ref, bo_ref, g1_ref, b1_ref, W1_ref, bW1_ref,
              W2_ref, bW2_ref, g2_ref, b2_ref, y_ref):
    x = out_ref[...] + jnp.dot(attn_ref[...], Wo_ref[...],
                               preferred_element_type=jnp.float32) + bo_ref[...]
    x = _ln(x, g1_ref[...], b1_ref[...])
    h = jnp.maximum(jnp.dot(x, W1_ref[...], preferred_element_type=jnp.float32)
                    + bW1_ref[...], 0.0)
    y = x + jnp.dot(h, W2_ref[...], preferred_element_type=jnp.float32) + bW2_ref[...]
    y_ref[...] = _ln(y, g2_ref[...], b2_ref[...])


def _tc2_call(out, attn, Wo, bo, g1, b1, W1, bW1, W2, bW2, g2, b2):
    full = lambda shape: pl.BlockSpec(shape, lambda i: (0,) * len(shape))
    return pl.pallas_call(
        _tc2_body,
        grid=(GRID,),
        in_specs=[
            pl.BlockSpec((BQ, DM), lambda i: (i, 0)),
            pl.BlockSpec((BQ, DM), lambda i: (i, 0)),
            full((DM, DM)), full((DM,)),
            full((DM,)), full((DM,)),
            full((DM, DFF)), full((DFF,)),
            full((DFF, DM)), full((DM,)),
            full((DM,)), full((DM,)),
        ],
        out_specs=pl.BlockSpec((BQ, DM), lambda i: (i, 0)),
        out_shape=jax.ShapeDtypeStruct((LQ, DM), jnp.float32),
    )(out, attn, Wo, bo, g1, b1, W1, bW1, W2, bW2, g2, b2)


def kernel(src0, src1, src2, pos0, pos1, pos2, level_embed, Wv, bv, Woff, boff,
           Wa, ba, Wo, bo, g1, b1, W1, bW1, W2, bW2, g2, b2):
    srcs = (src0, src1, src2)
    poss = (pos0, pos1, pos2)
    sf = []
    pf = []
    for lvl in range(NL):
        n = srcs[lvl].shape[2] * srcs[lvl].shape[3] * srcs[lvl].shape[4]
        sf.append(srcs[lvl].reshape(DM, n).T)
        pf.append(poss[lvl].reshape(DM, n).T + level_embed[lvl][None])
    out = jnp.concatenate(sf, 0)
    posle = jnp.concatenate(pf, 0)
    refq = jnp.asarray(_refq_np)

    perm = jnp.asarray(_perm)
    for l in range(3):
        Woff_r = Woff[l][:, perm]
        boff_r = boff[l][perm]
        value, idx3, wgt3 = _tc1_call(out, posle, refq, Wv[l], bv[l],
                                      Woff_r, boff_r, Wa[l], ba[l])
        attn = _sc_call(value.reshape(LQ * NH, HD),
                        idx3.reshape(LQ, NCORN * NCOL),
                        wgt3.reshape(LQ, NCORN * NCOL))
        out = _tc2_call(out, attn, Wo[l], bo[l], g1[l], b1[l], W1[l], bW1[l],
                        W2[l], bW2[l], g2[l], b2[l])
    return out[None]


# speedup vs baseline: 1.3414x; 1.3414x over previous
import functools

import numpy as np
import jax
import jax.numpy as jnp
from jax import lax
from jax.experimental import pallas as pl
from jax.experimental.pallas import tpu as pltpu, tpu_sc as plsc

SHAPES = ((8, 32, 32), (4, 16, 16), (2, 8, 8))
DM = 256
NH = 8
NL = 3
NP = 4
DFF = 1024
HD = DM // NH
LQ = sum(d * h * w for d, h, w in SHAPES)
NCOL = NH * NL * NP
NCORN = 8
BQ = 128
GRID = LQ // BQ
NWORK = 32
QPW = LQ // NWORK

_lvl = np.array([(k % 12) // 4 for k in range(NCOL)])
_m = np.array([k // 12 for k in range(NCOL)])
_w_np = np.array([SHAPES[l][2] for l in _lvl], np.float32)[None]
_h_np = np.array([SHAPES[l][1] for l in _lvl], np.float32)[None]
_d_np = np.array([SHAPES[l][0] for l in _lvl], np.float32)[None]
_starts = np.cumsum([0] + [d * h * w for d, h, w in SHAPES])[:3]
_lvlstart_np = np.array([_starts[l] for l in _lvl], np.float32)[None]
_m_np = _m.astype(np.float32)[None]
_S_np = np.kron(np.eye(NH, dtype=np.float32), np.ones((NL * NP, NL * NP), np.float32))

_ref_list = []
for d_, h_, w_ in SHAPES:
    zz, yy, xx = np.meshgrid(np.arange(d_) + 0.5, np.arange(h_) + 0.5,
                             np.arange(w_) + 0.5, indexing="ij")
    _ref_list.append(np.stack([(zz / d_).ravel(), (yy / h_).ravel(),
                               (xx / w_).ravel()], -1))
_refq_np = np.concatenate(_ref_list, 0).astype(np.float32)

_perm = np.empty(3 * NCOL, np.int64)
for m in range(NH):
    for l in range(NL):
        for p in range(NP):
            for c in range(3):
                _perm[c * NCOL + m * 12 + l * 4 + p] = ((m * 3 + l) * 4 + p) * 3 + c


def _tc1_body(out_ref, posle_ref, refq_ref, Wv_ref, bv_ref, Woff_ref, boff_ref,
              Wa_ref, ba_ref, S_ref, cst_ref, value_ref, idx_ref, wgt_ref):
    x = out_ref[...]
    q = x + posle_ref[...]
    value_ref[...] = jnp.dot(x, Wv_ref[...], preferred_element_type=jnp.float32) + bv_ref[...]

    offr = jnp.dot(q, Woff_ref[...], preferred_element_type=jnp.float32) + boff_ref[...]
    a = jnp.dot(q, Wa_ref[...], preferred_element_type=jnp.float32) + ba_ref[...]
    a = a - jnp.max(a, axis=-1, keepdims=True)
    e = jnp.exp(a)
    ssum = jnp.dot(e, S_ref[...], preferred_element_type=jnp.float32)
    aw = e / ssum

    w96 = cst_ref[0:1, :]
    h96 = cst_ref[1:2, :]
    d96 = cst_ref[2:3, :]
    lvlstart = cst_ref[3:4, :]
    m96 = cst_ref[4:5, :]

    r0 = refq_ref[:, 0:1]
    r1 = refq_ref[:, 1:2]
    r2 = refq_ref[:, 2:3]
    ix = r0 * w96 + offr[:, 0:NCOL] - 0.5
    iy = r1 * h96 + offr[:, NCOL:2 * NCOL] - 0.5
    iz = r2 * d96 + offr[:, 2 * NCOL:3 * NCOL] - 0.5
    x0 = jnp.floor(ix)
    y0 = jnp.floor(iy)
    z0 = jnp.floor(iz)

    idx_parts = []
    wgt_parts = []
    for dz in (0, 1):
        for dy in (0, 1):
            for dx in (0, 1):
                xi = x0 + dx
                yi = y0 + dy
                zi = z0 + dz
                wx = 1.0 - jnp.abs(ix - xi)
                wy = 1.0 - jnp.abs(iy - yi)
                wz = 1.0 - jnp.abs(iz - zi)
                valid = ((xi >= 0) & (xi <= w96 - 1) & (yi >= 0) & (yi <= h96 - 1)
                         & (zi >= 0) & (zi <= d96 - 1))
                w = wx * wy * wz * aw * valid.astype(jnp.float32)
                vox = (jnp.clip(zi, 0, d96 - 1) * h96 + jnp.clip(yi, 0, h96 - 1)) * w96 \
                    + jnp.clip(xi, 0, w96 - 1)
                row = (lvlstart + vox) * float(NH) + m96
                idx_parts.append(row.astype(jnp.int32)[:, None, :])
                wgt_parts.append(w[:, None, :])
    idx_ref[...] = jnp.concatenate(idx_parts, axis=1)
    wgt_ref[...] = jnp.concatenate(wgt_parts, axis=1)


_cst_np = np.concatenate([_w_np, _h_np, _d_np, _lvlstart_np, _m_np], 0)


def _tc1_call(out, posle, refq, Wv, bv, Woff_r, boff_r, Wa, ba):
    full = lambda shape: pl.BlockSpec(shape, lambda i: (0,) * len(shape))
    return pl.pallas_call(
        _tc1_body,
        grid=(GRID,),
        in_specs=[
            pl.BlockSpec((BQ, DM), lambda i: (i, 0)),
            pl.BlockSpec((BQ, DM), lambda i: (i, 0)),
            pl.BlockSpec((BQ, 3), lambda i: (i, 0)),
            full((DM, DM)), full((DM,)),
            full((DM, 3 * NCOL)), full((3 * NCOL,)),
            full((DM, NCOL)), full((NCOL,)),
            full((NCOL, NCOL)), full((5, NCOL)),
        ],
        out_specs=[
            pl.BlockSpec((BQ, DM), lambda i: (i, 0)),
            pl.BlockSpec((BQ, NCORN, NCOL), lambda i: (i, 0, 0)),
            pl.BlockSpec((BQ, NCORN, NCOL), lambda i: (i, 0, 0)),
        ],
        out_shape=[
            jax.ShapeDtypeStruct((LQ, DM), jnp.float32),
            jax.ShapeDtypeStruct((LQ, NCORN, NCOL), jnp.int32),
            jax.ShapeDtypeStruct((LQ, NCORN, NCOL), jnp.float32),
        ],
    )(out, posle, refq, Wv, bv, Woff_r, boff_r, Wa, ba,
      jnp.asarray(_S_np), jnp.asarray(_cst_np))


NROW = NCORN * NCOL


def _sc_body(value_hbm, idx_hbm, wgt_hbm, out_hbm,
             idx0, idx1, wgt0, wgt1, rows0, rows1, outv0, outv1,
             gs0, gs1, is0, is1, ws0, ws1, os0, os1):
    wid = lax.axis_index("s") * 2 + lax.axis_index("c")
    base = wid * QPW

    def prefetch(idxv, wgtv, isem, wsem, qq):
        pltpu.async_copy(idx_hbm.at[qq], idxv, isem)
        pltpu.async_copy(wgt_hbm.at[qq], wgtv.at[pl.ds(0, NROW)], wsem)

    def start_gather(idxv, isem, rowsv, gsem, qq):
        pltpu.make_async_copy(idx_hbm.at[qq], idxv, isem).wait()
        pltpu.async_copy(value_hbm.at[idxv], rowsv, gsem)

    def wait_gather(idxv, rowsv, gsem):
        pltpu.make_async_copy(value_hbm.at[idxv], rowsv, gsem).wait()

    def compute(wgtv, wsem, rowsv, outv, osem, qq, first):
        pltpu.make_async_copy(wgt_hbm.at[qq], wgtv.at[pl.ds(0, NROW)], wsem).wait()

        @pl.when(jnp.logical_not(first))
        def _():
            pltpu.make_async_copy(outv, out_hbm.at[qq], osem).wait()

        for m in range(NH):
            def cbody(c, acc):
                a0, a1 = acc
                jb = c * NCOL + m * 12
                wvec = wgtv[pl.ds(jb, 16)]
                for t in range(12):
                    j = jb + t
                    w = wvec[t]
                    a0 = a0 + w * rowsv[j, pl.ds(0, 16)]
                    a1 = a1 + w * rowsv[j, pl.ds(16, 16)]
                return (a0, a1)
            z = jnp.zeros((16,), jnp.float32)
            a0, a1 = lax.fori_loop(0, NCORN, cbody, (z, z))
            outv[pl.ds(m * 32, 16)] = a0
            outv[pl.ds(m * 32 + 16, 16)] = a1
        pltpu.async_copy(outv, out_hbm.at[qq], osem)

    prefetch(idx0, wgt0, is0, ws0, base)
    start_gather(idx0, is0, rows0, gs0, base)
    prefetch(idx1, wgt1, is1, ws1, base + 1)

    def ibody(i, carry):
        q0 = base + 2 * i
        first = i == 0
        not_last = i < QPW // 2 - 1
        wait_gather(idx0, rows0, gs0)
        start_gather(idx1, is1, rows1, gs1, q0 + 1)
        compute(wgt0, ws0, rows0, outv0, os0, q0, first)

        @pl.when(not_last)
        def _():
            prefetch(idx0, wgt0, is0, ws0, q0 + 2)

        wait_gather(idx1, rows1, gs1)
        compute(wgt1, ws1, rows1, outv1, os1, q0 + 1, first)

        @pl.when(not_last)
        def _():
            start_gather(idx0, is0, rows0, gs0, q0 + 2)
            prefetch(idx1, wgt1, is1, ws1, q0 + 3)

        return carry

    lax.fori_loop(0, QPW // 2, ibody, 0)
    pltpu.make_async_copy(outv0, out_hbm.at[base], os0).wait()
    pltpu.make_async_copy(outv1, out_hbm.at[base + 1], os1).wait()


@functools.partial(
    pl.kernel,
    out_type=jax.ShapeDtypeStruct((LQ, DM), jnp.float32),
    mesh=plsc.VectorSubcoreMesh(core_axis_name="c", subcore_axis_name="s",
                                num_cores=2, num_subcores=16),
    compiler_params=pltpu.CompilerParams(needs_layout_passes=False,
                                         use_tc_tiling_on_sc=False),
    scratch_types=[
        pltpu.VMEM((NROW,), jnp.int32),
        pltpu.VMEM((NROW,), jnp.int32),
        pltpu.VMEM((NROW + 16,), jnp.float32),
        pltpu.VMEM((NROW + 16,), jnp.float32),
        pltpu.VMEM((NROW, HD), jnp.float32),
        pltpu.VMEM((NROW, HD), jnp.float32),
        pltpu.VMEM((DM,), jnp.float32),
        pltpu.VMEM((DM,), jnp.float32),
        pltpu.SemaphoreType.DMA,
        pltpu.SemaphoreType.DMA,
        pltpu.SemaphoreType.DMA,
        pltpu.SemaphoreType.DMA,
        pltpu.SemaphoreType.DMA,
        pltpu.SemaphoreType.DMA,
        pltpu.SemaphoreType.DMA,
        pltpu.SemaphoreType.DMA,
    ],
)
def _sc_call(value_hbm, idx_hbm, wgt_hbm, out_hbm,
             idx0, idx1, wgt0, wgt1, rows0, rows1, outv0, outv1,
             gs0, gs1, is0, is1, ws0, ws1, os0, os1):
    _sc_body(value_hbm, idx_hbm, wgt_hbm, out_hbm,
             idx0, idx1, wgt0, wgt1, rows0, rows1, outv0, outv1,
             gs0, gs1, is0, is1, ws0, ws1, os0, os1)


def _ln(x, g, b):
    m = jnp.mean(x, axis=-1, keepdims=True)
    v = jnp.mean((x - m) ** 2, axis=-1, keepdims=True)
    return (x - m) * lax.rsqrt(v + 1e-5) * g + b


def _tc2_body(out_ref, attn_ref, Wo_ref, bo_ref, g1_ref, b1_ref, W1_ref, bW1_ref,
              W2_ref, bW2_ref, g2_ref, b2_ref, y_ref):
    x = out_ref[...] + jnp.dot(attn_ref[...], Wo_ref[...],
                               preferred_element_type=jnp.float32) + bo_ref[...]
    x = _ln(x, g1_ref[...], b1_ref[...])
    h = jnp.maximum(jnp.dot(x, W1_ref[...], preferred_element_type=jnp.float32)
                    + bW1_ref[...], 0.0)
    y = x + jnp.dot(h, W2_ref[...], preferred_element_type=jnp.float32) + bW2_ref[...]
    y_ref[...] = _ln(y, g2_ref[...], b2_ref[...])


def _tc2_call(out, attn, Wo, bo, g1, b1, W1, bW1, W2, bW2, g2, b2):
    full = lambda shape: pl.BlockSpec(shape, lambda i: (0,) * len(shape))
    return pl.pallas_call(
        _tc2_body,
        grid=(GRID,),
        in_specs=[
            pl.BlockSpec((BQ, DM), lambda i: (i, 0)),
            pl.BlockSpec((BQ, DM), lambda i: (i, 0)),
            full((DM, DM)), full((DM,)),
            full((DM,)), full((DM,)),
            full((DM, DFF)), full((DFF,)),
            full((DFF, DM)), full((DM,)),
            full((DM,)), full((DM,)),
        ],
        out_specs=pl.BlockSpec((BQ, DM), lambda i: (i, 0)),
        out_shape=jax.ShapeDtypeStruct((LQ, DM), jnp.float32),
    )(out, attn, Wo, bo, g1, b1, W1, bW1, W2, bW2, g2, b2)


def kernel(src0, src1, src2, pos0, pos1, pos2, level_embed, Wv, bv, Woff, boff,
           Wa, ba, Wo, bo, g1, b1, W1, bW1, W2, bW2, g2, b2):
    srcs = (src0, src1, src2)
    poss = (pos0, pos1, pos2)
    sf = []
    pf = []
    for lvl in range(NL):
        n = srcs[lvl].shape[2] * srcs[lvl].shape[3] * srcs[lvl].shape[4]
        sf.append(srcs[lvl].reshape(DM, n).T)
        pf.append(poss[lvl].reshape(DM, n).T + level_embed[lvl][None])
    out = jnp.concatenate(sf, 0)
    posle = jnp.concatenate(pf, 0)
    refq = jnp.asarray(_refq_np)

    perm = jnp.asarray(_perm)
    for l in range(3):
        Woff_r = Woff[l][:, perm]
        boff_r = boff[l][perm]
        value, idx3, wgt3 = _tc1_call(out, posle, refq, Wv[l], bv[l],
                                      Woff_r, boff_r, Wa[l], ba[l])
        attn = _sc_call(value.reshape(LQ * NH, HD),
                        idx3.reshape(LQ, NCORN * NCOL),
                        wgt3.reshape(LQ, NCORN * NCOL))
        out = _tc2_call(out, attn, Wo[l], bo[l], g1[l], b1[l], W1[l], bW1[l],
                        W2[l], bW2[l], g2[l], b2[l])
    return out[None]
